# baseline (speedup 1.0000x reference)
import jax
import jax.numpy as jnp
from jax import lax
from jax.experimental import pallas as pl
from jax.experimental.pallas import tpu as pltpu

_TOP_N = 2
_REG_W = 0.05
_RBLK = 256
_CBLK = 512
_KEY_LO = 1234


def _threefry_bits(x1_init):
    ks0 = 0
    ks1 = _KEY_LO
    ks2 = ks0 ^ ks1 ^ 0x1BD11BDA
    rot_a = (13, 15, 26, 6)
    rot_b = (17, 29, 16, 24)

    def rounds(x0, x1, rots):
        for r in rots:
            x0 = x0 + x1
            x1 = ((x1 << r) | (x1 >> (32 - r))) ^ x0
        return x0, x1

    def u32(v):
        return jnp.uint32(v & 0xFFFFFFFF)

    x0 = jnp.zeros_like(x1_init) + u32(ks0)
    x1 = x1_init
    x0, x1 = rounds(x0, x1, rot_a)
    x0 = x0 + u32(ks1)
    x1 = x1 + u32(ks2 + 1)
    x0, x1 = rounds(x0, x1, rot_b)
    x0 = x0 + u32(ks2)
    x1 = x1 + u32(ks0 + 2)
    x0, x1 = rounds(x0, x1, rot_a)
    x0 = x0 + u32(ks0)
    x1 = x1 + u32(ks1 + 3)
    x0, x1 = rounds(x0, x1, rot_b)
    x0 = x0 + u32(ks1)
    x1 = x1 + u32(ks2 + 4)
    x0, x1 = rounds(x0, x1, rot_a)
    x0 = x0 + u32(ks2)
    x1 = x1 + u32(ks0 + 5)
    return x0 ^ x1


def _body(yp_ref, lp_ref, ep_ref, rid_ref,
          ls2_ref, ys2_ref, cid2_ref,
          out_ref,
          lossacc, regacc, e2, ay2):
    ntiles = ls2_ref.shape[0]
    n = ntiles * ls2_ref.shape[1]
    k = pl.program_id(0)
    nsteps = pl.num_programs(0)
    r0 = k * _RBLK

    max_y = jnp.max(ys2_ref[...])

    @pl.when(k == 0)
    def _init():
        lossacc[...] = jnp.zeros_like(lossacc)
        regacc[...] = jnp.zeros_like(regacc)
        e2[...] = jnp.exp(ys2_ref[...] - max_y)
        ay2[...] = jnp.abs(ys2_ref[...])

    e_c = ep_ref[pl.ds(r0, _RBLK), :]

    @pl.when(jnp.max(e_c) > 0.0)
    def _block():
        y_c = yp_ref[pl.ds(r0, _RBLK), :]
        l_c = lp_ref[pl.ds(r0, _RBLK), :]
        rid = rid_ref[pl.ds(r0, _RBLK), :]
        ridn = lax.bitcast_convert_type(rid * n + _KEY_LO, jnp.uint32)
        l_eff = jnp.where(e_c > 0.0, l_c, jnp.inf)

        min_len = jnp.min(l_eff)
        start_t = jnp.int32(0)
        for t in range(ntiles):
            tile_max = jnp.max(ls2_ref[t, :])
            start_t += jnp.where(tile_max - min_len > 0.0, 0, 1).astype(
                jnp.int32)

        cols_loc = lax.broadcasted_iota(jnp.int32, (_RBLK, _CBLK), 1)
        zero_c = jnp.zeros((_RBLK, 1), jnp.float32)

        def extract(v, e_tile, ay_tile):
            m = jnp.max(v, axis=1, keepdims=True)
            ism = v == m
            pos = jnp.min(jnp.where(ism, cols_loc, _CBLK), axis=1,
                          keepdims=True)
            sel = cols_loc == pos
            ev = jnp.sum(jnp.where(sel, e_tile, 0.0), axis=1, keepdims=True)
            av = jnp.sum(jnp.where(sel, ay_tile, 0.0), axis=1, keepdims=True)
            return m, ev, av, jnp.where(sel, -1.0, v)

        def insert(state, x, ex, ax):
            v1, e1, a1, v2, e2_, a2, v3 = state
            gt1 = x > v1
            gt2 = x > v2
            gt3 = x > v3
            v3n = jnp.where(gt2, v2, jnp.where(gt3, x, v3))
            v2n = jnp.where(gt1, v1, jnp.where(gt2, x, v2))
            e2n = jnp.where(gt1, e1, jnp.where(gt2, ex, e2_))
            a2n = jnp.where(gt1, a1, jnp.where(gt2, ax, a2))
            v1n = jnp.where(gt1, x, v1)
            e1n = jnp.where(gt1, ex, e1)
            a1n = jnp.where(gt1, ax, a1)
            return (v1n, e1n, a1n, v2n, e2n, a2n, v3n)

        def tile_body(t, state):
            l_tile = ls2_ref[pl.ds(t, 1), :]
            cid = cid2_ref[pl.ds(t, 1), :]
            e_tile = e2[pl.ds(t, 1), :]
            ay_tile = ay2[pl.ds(t, 1), :]
            x1_init = ridn + lax.bitcast_convert_type(cid, jnp.uint32)
            bits = _threefry_bits(x1_init)
            f = lax.bitcast_convert_type(
                (bits >> 9) | jnp.uint32(0x3F800000), jnp.float32)
            pair = (l_tile - l_eff) > 0.0
            val = jnp.where(pair, f, 0.0)
            t1, ev1, av1, val = extract(val, e_tile, ay_tile)
            t2, ev2, av2, val = extract(val, e_tile, ay_tile)
            t3 = jnp.max(val, axis=1, keepdims=True)
            state = insert(state, t1, ev1, av1)
            state = insert(state, t2, ev2, av2)
            state = insert(state, t3, zero_c, zero_c)
            return state

        init = (zero_c, zero_c, zero_c, zero_c, zero_c, zero_c, zero_c)
        v1, e1, a1, v2, e2p, a2, v3 = lax.fori_loop(
            start_t, ntiles, tile_body, init)

        s1 = (v1 > v3).astype(jnp.float32)
        s2 = (v2 > v3).astype(jnp.float32)
        validf = s1

        row_sum = s1 * e1 + s2 * e2p + validf * jnp.exp(y_c - max_y)
        rs_safe = jnp.where(validf > 0.0, row_sum, 1.0)
        row_loss = validf * ((max_y - y_c) + jnp.log(rs_safe))
        row_reg = s1 * a1 + s2 * a2 + validf * jnp.abs(y_c)
        lossacc[...] += jnp.sum(row_loss, keepdims=True)[:1, :1]
        regacc[...] += jnp.sum(row_reg, keepdims=True)[:1, :1]

    @pl.when(k == nsteps - 1)
    def _finish():
        out_ref[...] = lossacc[...] + _REG_W * regacc[...]


def _build_call(n, interpret=False):
    ntiles = n // _CBLK
    full_col = pl.BlockSpec((n, 1), lambda k: (0, 0))
    full_t = pl.BlockSpec((ntiles, _CBLK), lambda k: (0, 0))
    return pl.pallas_call(
        _body,
        grid=(n // _RBLK,),
        in_specs=[full_col, full_col, full_col, full_col,
                  full_t, full_t, full_t],
        out_specs=pl.BlockSpec((1, 1), lambda k: (0, 0)),
        out_shape=jax.ShapeDtypeStruct((1, 1), jnp.float32),
        scratch_shapes=[
            pltpu.VMEM((1, 1), jnp.float32),
            pltpu.VMEM((1, 1), jnp.float32),
            pltpu.VMEM((ntiles, _CBLK), jnp.float32),
            pltpu.VMEM((ntiles, _CBLK), jnp.float32),
        ],
        compiler_params=pltpu.CompilerParams(
            dimension_semantics=("arbitrary",)),
        interpret=interpret,
    )


def _prep(y_pred, length, event):
    n = y_pred.shape[0]
    iota = lax.iota(jnp.int32, n)
    row_key = jnp.where(event > 0.0, length, jnp.inf)
    _, rid_v, y_pv, l_pv, e_pv = lax.sort(
        (row_key, iota, y_pred, length, event), num_keys=1)
    y_p = y_pv.reshape(n, 1)
    l_p = l_pv.reshape(n, 1)
    e_p = e_pv.reshape(n, 1)
    rid = rid_v.reshape(n, 1)
    ntiles = n // _CBLK
    l_sv, cid_v, y_sv = lax.sort((length, iota, y_pred), num_keys=1)
    ls2 = l_sv.reshape(ntiles, _CBLK)
    ys2 = y_sv.reshape(ntiles, _CBLK)
    cid2 = cid_v.reshape(ntiles, _CBLK)
    return y_p, l_p, e_p, rid, ls2, ys2, cid2


def kernel(y_pred, length, event):
    n = y_pred.shape[0]
    out = _build_call(n)(*_prep(y_pred, length, event))
    return out[0, 0]


# speedup vs baseline: 1.0614x; 1.0614x over previous
import jax
import jax.numpy as jnp
from jax import lax
from jax.experimental import pallas as pl
from jax.experimental.pallas import tpu as pltpu

_TOP_N = 2
_REG_W = 0.05
_RBLK = 256
_CBLK = 1024
_KEY_LO = 1234


def _threefry_bits(x1_init):
    ks0 = 0
    ks1 = _KEY_LO
    ks2 = ks0 ^ ks1 ^ 0x1BD11BDA
    rot_a = (13, 15, 26, 6)
    rot_b = (17, 29, 16, 24)

    def rounds(x0, x1, rots):
        for r in rots:
            x0 = x0 + x1
            x1 = ((x1 << r) | (x1 >> (32 - r))) ^ x0
        return x0, x1

    def u32(v):
        return jnp.uint32(v & 0xFFFFFFFF)

    x0 = jnp.zeros_like(x1_init) + u32(ks0)
    x1 = x1_init
    x0, x1 = rounds(x0, x1, rot_a)
    x0 = x0 + u32(ks1)
    x1 = x1 + u32(ks2 + 1)
    x0, x1 = rounds(x0, x1, rot_b)
    x0 = x0 + u32(ks2)
    x1 = x1 + u32(ks0 + 2)
    x0, x1 = rounds(x0, x1, rot_a)
    x0 = x0 + u32(ks0)
    x1 = x1 + u32(ks1 + 3)
    x0, x1 = rounds(x0, x1, rot_b)
    x0 = x0 + u32(ks1)
    x1 = x1 + u32(ks2 + 4)
    x0, x1 = rounds(x0, x1, rot_a)
    x0 = x0 + u32(ks2)
    x1 = x1 + u32(ks0 + 5)
    return x0 ^ x1


def _body(yp_ref, lp_ref, ep_ref, rid_ref,
          ls2_ref, ys2_ref, cid2_ref,
          out_ref,
          lossacc, regacc, e2, ay2):
    ntiles = ls2_ref.shape[0]
    n = ntiles * ls2_ref.shape[1]
    k = pl.program_id(0)
    nsteps = pl.num_programs(0)
    r0 = k * _RBLK

    max_y = jnp.max(ys2_ref[...])

    @pl.when(k == 0)
    def _init():
        lossacc[...] = jnp.zeros_like(lossacc)
        regacc[...] = jnp.zeros_like(regacc)
        e2[...] = jnp.exp(ys2_ref[...] - max_y)
        ay2[...] = jnp.abs(ys2_ref[...])

    e_c = ep_ref[pl.ds(r0, _RBLK), :]

    @pl.when(jnp.max(e_c) > 0.0)
    def _block():
        y_c = yp_ref[pl.ds(r0, _RBLK), :]
        l_c = lp_ref[pl.ds(r0, _RBLK), :]
        rid = rid_ref[pl.ds(r0, _RBLK), :]
        ridn = lax.bitcast_convert_type(rid * n + _KEY_LO, jnp.uint32)
        l_eff = jnp.where(e_c > 0.0, l_c, jnp.inf)

        min_len = jnp.min(l_eff)
        start_t = jnp.int32(0)
        for t in range(ntiles):
            tile_max = jnp.max(ls2_ref[t, :])
            start_t += jnp.where(tile_max - min_len > 0.0, 0, 1).astype(
                jnp.int32)

        cols_loc = lax.broadcasted_iota(jnp.int32, (_RBLK, _CBLK), 1)
        zero_c = jnp.zeros((_RBLK, 1), jnp.float32)

        def extract(v, e_tile, ay_tile):
            m = jnp.max(v, axis=1, keepdims=True)
            ism = v == m
            pos = jnp.min(jnp.where(ism, cols_loc, _CBLK), axis=1,
                          keepdims=True)
            sel = cols_loc == pos
            ev = jnp.sum(jnp.where(sel, e_tile, 0.0), axis=1, keepdims=True)
            av = jnp.sum(jnp.where(sel, ay_tile, 0.0), axis=1, keepdims=True)
            return m, ev, av, jnp.where(sel, -1.0, v)

        def insert(state, x, ex, ax):
            v1, e1, a1, v2, e2_, a2, v3 = state
            gt1 = x > v1
            gt2 = x > v2
            gt3 = x > v3
            v3n = jnp.where(gt2, v2, jnp.where(gt3, x, v3))
            v2n = jnp.where(gt1, v1, jnp.where(gt2, x, v2))
            e2n = jnp.where(gt1, e1, jnp.where(gt2, ex, e2_))
            a2n = jnp.where(gt1, a1, jnp.where(gt2, ax, a2))
            v1n = jnp.where(gt1, x, v1)
            e1n = jnp.where(gt1, ex, e1)
            a1n = jnp.where(gt1, ax, a1)
            return (v1n, e1n, a1n, v2n, e2n, a2n, v3n)

        def tile_body(t, state):
            l_tile = ls2_ref[pl.ds(t, 1), :]
            cid = cid2_ref[pl.ds(t, 1), :]
            e_tile = e2[pl.ds(t, 1), :]
            ay_tile = ay2[pl.ds(t, 1), :]
            x1_init = ridn + lax.bitcast_convert_type(cid, jnp.uint32)
            bits = _threefry_bits(x1_init)
            f = lax.bitcast_convert_type(
                (bits >> 9) | jnp.uint32(0x3F800000), jnp.float32)
            pair = (l_tile - l_eff) > 0.0
            val = jnp.where(pair, f, 0.0)
            t1, ev1, av1, val = extract(val, e_tile, ay_tile)
            t2, ev2, av2, val = extract(val, e_tile, ay_tile)
            t3 = jnp.max(val, axis=1, keepdims=True)
            state = insert(state, t1, ev1, av1)
            state = insert(state, t2, ev2, av2)
            state = insert(state, t3, zero_c, zero_c)
            return state

        init = (zero_c, zero_c, zero_c, zero_c, zero_c, zero_c, zero_c)
        v1, e1, a1, v2, e2p, a2, v3 = lax.fori_loop(
            start_t, ntiles, tile_body, init)

        s1 = (v1 > v3).astype(jnp.float32)
        s2 = (v2 > v3).astype(jnp.float32)
        validf = s1

        row_sum = s1 * e1 + s2 * e2p + validf * jnp.exp(y_c - max_y)
        rs_safe = jnp.where(validf > 0.0, row_sum, 1.0)
        row_loss = validf * ((max_y - y_c) + jnp.log(rs_safe))
        row_reg = s1 * a1 + s2 * a2 + validf * jnp.abs(y_c)
        lossacc[...] += jnp.sum(row_loss, keepdims=True)[:1, :1]
        regacc[...] += jnp.sum(row_reg, keepdims=True)[:1, :1]

    @pl.when(k == nsteps - 1)
    def _finish():
        out_ref[...] = lossacc[...] + _REG_W * regacc[...]


def _build_call(n, interpret=False):
    ntiles = n // _CBLK
    full_col = pl.BlockSpec((n, 1), lambda k: (0, 0))
    full_t = pl.BlockSpec((ntiles, _CBLK), lambda k: (0, 0))
    return pl.pallas_call(
        _body,
        grid=(n // _RBLK,),
        in_specs=[full_col, full_col, full_col, full_col,
                  full_t, full_t, full_t],
        out_specs=pl.BlockSpec((1, 1), lambda k: (0, 0)),
        out_shape=jax.ShapeDtypeStruct((1, 1), jnp.float32),
        scratch_shapes=[
            pltpu.VMEM((1, 1), jnp.float32),
            pltpu.VMEM((1, 1), jnp.float32),
            pltpu.VMEM((ntiles, _CBLK), jnp.float32),
            pltpu.VMEM((ntiles, _CBLK), jnp.float32),
        ],
        compiler_params=pltpu.CompilerParams(
            dimension_semantics=("arbitrary",)),
        interpret=interpret,
    )


def _prep(y_pred, length, event):
    n = y_pred.shape[0]
    iota = lax.iota(jnp.int32, n)
    row_key = jnp.where(event > 0.0, length, jnp.inf)
    _, rid_v, y_pv, l_pv, e_pv = lax.sort(
        (row_key, iota, y_pred, length, event), num_keys=1)
    y_p = y_pv.reshape(n, 1)
    l_p = l_pv.reshape(n, 1)
    e_p = e_pv.reshape(n, 1)
    rid = rid_v.reshape(n, 1)
    ntiles = n // _CBLK
    l_sv, cid_v, y_sv = lax.sort((length, iota, y_pred), num_keys=1)
    ls2 = l_sv.reshape(ntiles, _CBLK)
    ys2 = y_sv.reshape(ntiles, _CBLK)
    cid2 = cid_v.reshape(ntiles, _CBLK)
    return y_p, l_p, e_p, rid, ls2, ys2, cid2


def kernel(y_pred, length, event):
    n = y_pred.shape[0]
    out = _build_call(n)(*_prep(y_pred, length, event))
    return out[0, 0]


# speedup vs baseline: 1.0717x; 1.0097x over previous
import jax
import jax.numpy as jnp
from jax import lax
from jax.experimental import pallas as pl
from jax.experimental.pallas import tpu as pltpu

_TOP_N = 2
_REG_W = 0.05
_RBLK = 256
_CBLK = 1024
_KEY_LO = 1234


def _threefry_bits(x1_init):
    ks0 = 0
    ks1 = _KEY_LO
    ks2 = ks0 ^ ks1 ^ 0x1BD11BDA
    rot_a = (13, 15, 26, 6)
    rot_b = (17, 29, 16, 24)

    def rounds(x0, x1, rots):
        for r in rots:
            x0 = x0 + x1
            x1 = ((x1 << r) | (x1 >> (32 - r))) ^ x0
        return x0, x1

    def u32(v):
        return jnp.uint32(v & 0xFFFFFFFF)

    x0 = jnp.zeros_like(x1_init) + u32(ks0)
    x1 = x1_init
    x0, x1 = rounds(x0, x1, rot_a)
    x0 = x0 + u32(ks1)
    x1 = x1 + u32(ks2 + 1)
    x0, x1 = rounds(x0, x1, rot_b)
    x0 = x0 + u32(ks2)
    x1 = x1 + u32(ks0 + 2)
    x0, x1 = rounds(x0, x1, rot_a)
    x0 = x0 + u32(ks0)
    x1 = x1 + u32(ks1 + 3)
    x0, x1 = rounds(x0, x1, rot_b)
    x0 = x0 + u32(ks1)
    x1 = x1 + u32(ks2 + 4)
    x0, x1 = rounds(x0, x1, rot_a)
    x0 = x0 + u32(ks2)
    x1 = x1 + u32(ks0 + 5)
    return x0 ^ x1


def _body(yp_ref, lp_ref, ep_ref, rid_ref,
          ls2_ref, cid2_ref,
          ysc_ref,
          out_ref,
          lossacc, regacc, ec, ayc):
    ntiles = ls2_ref.shape[0]
    n = ntiles * ls2_ref.shape[1]
    k = pl.program_id(0)
    nsteps = pl.num_programs(0)
    r0 = k * _RBLK

    max_y = jnp.max(ysc_ref[...])

    @pl.when(k == 0)
    def _init():
        lossacc[...] = jnp.zeros_like(lossacc)
        regacc[...] = jnp.zeros_like(regacc)
        ec[...] = jnp.exp(ysc_ref[...] - max_y)
        ayc[...] = jnp.abs(ysc_ref[...])

    e_c = ep_ref[pl.ds(r0, _RBLK), :]

    @pl.when(jnp.max(e_c) > 0.0)
    def _block():
        y_c = yp_ref[pl.ds(r0, _RBLK), :]
        l_c = lp_ref[pl.ds(r0, _RBLK), :]
        rid = rid_ref[pl.ds(r0, _RBLK), :]
        ridn = lax.bitcast_convert_type(rid * n + _KEY_LO, jnp.uint32)
        l_eff = jnp.where(e_c > 0.0, l_c, jnp.inf)

        min_len = jnp.min(l_eff)
        start_t = jnp.int32(0)
        for t in range(ntiles):
            tile_max = jnp.max(ls2_ref[t, :])
            start_t += jnp.where(tile_max - min_len > 0.0, 0, 1).astype(
                jnp.int32)

        cols_loc = lax.broadcasted_iota(jnp.int32, (_RBLK, _CBLK), 1)
        zero_c = jnp.zeros((_RBLK, 1), jnp.float32)

        def extract(v, pmat):
            m = jnp.max(v, axis=1, keepdims=True)
            ism = v == m
            pos = jnp.min(jnp.where(ism, cols_loc, _CBLK), axis=1,
                          keepdims=True)
            sel = cols_loc == pos
            eav = jax.lax.dot_general(
                sel.astype(jnp.float32), pmat,
                (((1,), (0,)), ((), ())),
                preferred_element_type=jnp.float32)
            return m, eav[:, 0:1], eav[:, 1:2], jnp.where(sel, -1.0, v)

        def insert(state, x, ex, ax):
            v1, e1, a1, v2, e2_, a2, v3 = state
            gt1 = x > v1
            gt2 = x > v2
            gt3 = x > v3
            v3n = jnp.where(gt2, v2, jnp.where(gt3, x, v3))
            v2n = jnp.where(gt1, v1, jnp.where(gt2, x, v2))
            e2n = jnp.where(gt1, e1, jnp.where(gt2, ex, e2_))
            a2n = jnp.where(gt1, a1, jnp.where(gt2, ax, a2))
            v1n = jnp.where(gt1, x, v1)
            e1n = jnp.where(gt1, ex, e1)
            a1n = jnp.where(gt1, ax, a1)
            return (v1n, e1n, a1n, v2n, e2n, a2n, v3n)

        def tile_body(t, state):
            l_tile = ls2_ref[pl.ds(t, 1), :]
            cid = cid2_ref[pl.ds(t, 1), :]
            c0 = t * _CBLK
            pmat = jnp.concatenate(
                [ec[pl.ds(c0, _CBLK), :], ayc[pl.ds(c0, _CBLK), :]],
                axis=1)
            x1_init = ridn + lax.bitcast_convert_type(cid, jnp.uint32)
            bits = _threefry_bits(x1_init)
            f = lax.bitcast_convert_type(
                (bits >> 9) | jnp.uint32(0x3F800000), jnp.float32)
            pair = (l_tile - l_eff) > 0.0
            val = jnp.where(pair, f, 0.0)
            t1, ev1, av1, val = extract(val, pmat)
            t2, ev2, av2, val = extract(val, pmat)
            t3 = jnp.max(val, axis=1, keepdims=True)
            state = insert(state, t1, ev1, av1)
            state = insert(state, t2, ev2, av2)
            state = insert(state, t3, zero_c, zero_c)
            return state

        init = (zero_c, zero_c, zero_c, zero_c, zero_c, zero_c, zero_c)
        v1, e1, a1, v2, e2p, a2, v3 = lax.fori_loop(
            start_t, ntiles, tile_body, init)

        s1 = (v1 > v3).astype(jnp.float32)
        s2 = (v2 > v3).astype(jnp.float32)
        validf = s1

        row_sum = s1 * e1 + s2 * e2p + validf * jnp.exp(y_c - max_y)
        rs_safe = jnp.where(validf > 0.0, row_sum, 1.0)
        row_loss = validf * ((max_y - y_c) + jnp.log(rs_safe))
        row_reg = s1 * a1 + s2 * a2 + validf * jnp.abs(y_c)
        lossacc[...] += jnp.sum(row_loss, keepdims=True)[:1, :1]
        regacc[...] += jnp.sum(row_reg, keepdims=True)[:1, :1]

    @pl.when(k == nsteps - 1)
    def _finish():
        out_ref[...] = lossacc[...] + _REG_W * regacc[...]


def _build_call(n, interpret=False):
    ntiles = n // _CBLK
    full_col = pl.BlockSpec((n, 1), lambda k: (0, 0))
    full_t = pl.BlockSpec((ntiles, _CBLK), lambda k: (0, 0))
    return pl.pallas_call(
        _body,
        grid=(n // _RBLK,),
        in_specs=[full_col, full_col, full_col, full_col,
                  full_t, full_t, full_col],
        out_specs=pl.BlockSpec((1, 1), lambda k: (0, 0)),
        out_shape=jax.ShapeDtypeStruct((1, 1), jnp.float32),
        scratch_shapes=[
            pltpu.VMEM((1, 1), jnp.float32),
            pltpu.VMEM((1, 1), jnp.float32),
            pltpu.VMEM((n, 1), jnp.float32),
            pltpu.VMEM((n, 1), jnp.float32),
        ],
        compiler_params=pltpu.CompilerParams(
            dimension_semantics=("arbitrary",)),
        interpret=interpret,
    )


def _prep(y_pred, length, event):
    n = y_pred.shape[0]
    iota = lax.iota(jnp.int32, n)
    row_key = jnp.where(event > 0.0, length, jnp.inf)
    _, rid_v, y_pv, l_pv, e_pv = lax.sort(
        (row_key, iota, y_pred, length, event), num_keys=1)
    y_p = y_pv.reshape(n, 1)
    l_p = l_pv.reshape(n, 1)
    e_p = e_pv.reshape(n, 1)
    rid = rid_v.reshape(n, 1)
    ntiles = n // _CBLK
    l_sv, cid_v, y_sv = lax.sort((length, iota, y_pred), num_keys=1)
    ls2 = l_sv.reshape(ntiles, _CBLK)
    cid2 = cid_v.reshape(ntiles, _CBLK)
    ysc = y_sv.reshape(n, 1)
    return y_p, l_p, e_p, rid, ls2, cid2, ysc


def kernel(y_pred, length, event):
    n = y_pred.shape[0]
    out = _build_call(n)(*_prep(y_pred, length, event))
    return out[0, 0]
